# baseline (speedup 1.0000x reference)
import functools

import jax
import jax.numpy as jnp
from jax import lax
from jax.experimental import pallas as pl
from jax.experimental.pallas import tpu as pltpu
from jax.experimental.pallas import tpu_sc as plsc

N = 10000
E = 320000
D = 128
H0 = 8
NB = 400
GRID = N // NB
NTILES = 16
NPAD = 10240
NPT = NPAD // NTILES
ZCH = 128
CH = 64
EPAD = 320512
EPT = EPAD // NTILES
NCHK = EPT // CH
NEG = -1e30


def _lrelu(x, slope):
    return jnp.where(x >= 0.0, x, x * slope)


def _tc1_body(x_ref, pw_ref, pb_ref, fw_ref, al_ref, ar_ref, rw_ref,
              tbl_ref, elf_ref, erf_ref, res_ref, hg_ref, cel_ref, cer_ref):
    i = pl.program_id(0)
    x = x_ref[...]
    h = jnp.dot(x, pw_ref[...], preferred_element_type=jnp.float32) + pb_ref[...]
    els = []
    ers = []
    for k in range(H0):
        fk = jnp.dot(h, fw_ref[:, k * D:(k + 1) * D],
                     preferred_element_type=jnp.float32)
        tbl_ref[k] = fk
        res_ref[k] = jnp.dot(h, rw_ref[:, k * D:(k + 1) * D],
                             preferred_element_type=jnp.float32)
        els.append(jnp.sum(fk * al_ref[:, k * D:(k + 1) * D], axis=-1,
                           keepdims=True))
        ers.append(jnp.sum(fk * ar_ref[:, k * D:(k + 1) * D], axis=-1,
                           keepdims=True))
    el = jnp.concatenate(els, axis=-1)
    er = jnp.concatenate(ers, axis=-1)
    elf_ref[...] = el
    erf_ref[...] = er

    @pl.when(i == 0)
    def _():
        hg_ref[...] = jnp.zeros_like(hg_ref)
        cel_ref[...] = jnp.full_like(cel_ref, NEG)
        cer_ref[...] = jnp.full_like(cer_ref, NEG)

    hg_ref[...] += jnp.sum(h, axis=0, keepdims=True)
    zpad = jnp.full((1, 8), NEG, jnp.float32)
    melp = jnp.concatenate([jnp.max(el, axis=0, keepdims=True), zpad], axis=-1)
    merp = jnp.concatenate([jnp.max(er, axis=0, keepdims=True), zpad], axis=-1)
    cel_ref[...] = jnp.maximum(cel_ref[...], melp)
    cer_ref[...] = jnp.maximum(cer_ref[...], merp)


def _tc1(x, proj_W, proj_b, fc0_W, attn_l0, attn_r0, res0_W):
    rep = lambda shape: pl.BlockSpec(shape, lambda i: tuple(0 for _ in shape))
    return pl.pallas_call(
        _tc1_body,
        grid=(GRID,),
        in_specs=[
            pl.BlockSpec((NB, D), lambda i: (i, 0)),
            rep((D, D)), rep((1, D)), rep((D, H0 * D)),
            rep((1, H0 * D)), rep((1, H0 * D)), rep((D, H0 * D)),
        ],
        out_specs=[
            pl.BlockSpec((H0, NB, D), lambda i: (0, i, 0)),
            pl.BlockSpec((NB, H0), lambda i: (i, 0)),
            pl.BlockSpec((NB, H0), lambda i: (i, 0)),
            pl.BlockSpec((H0, NB, D), lambda i: (0, i, 0)),
            rep((1, D)), rep((1, 16)), rep((1, 16)),
        ],
        out_shape=[
            jax.ShapeDtypeStruct((H0, N, D), jnp.float32),
            jax.ShapeDtypeStruct((N, H0), jnp.float32),
            jax.ShapeDtypeStruct((N, H0), jnp.float32),
            jax.ShapeDtypeStruct((H0, N, D), jnp.float32),
            jax.ShapeDtypeStruct((1, D), jnp.float32),
            jax.ShapeDtypeStruct((1, 16), jnp.float32),
            jax.ShapeDtypeStruct((1, 16), jnp.float32),
        ],
    )(x, proj_W, proj_b, fc0_W, attn_l0, attn_r0, res0_W)


@functools.lru_cache(maxsize=None)
def _sc_mesh():
    return plsc.VectorSubcoreMesh(core_axis_name="c", subcore_axis_name="s")


def _exp16(x):
    y = x * 1.4426950408889634
    k = y.astype(jnp.int32)
    k = jnp.where(y < k.astype(jnp.float32), k - 1, k)
    f = y - k.astype(jnp.float32)
    p = jnp.full((16,), 1.8775767e-3, jnp.float32)
    for coef in (8.9893397e-3, 5.5826318e-2, 2.4015361e-1, 6.9315308e-1, 1.0):
        p = p * f + coef
    kc = jnp.maximum(k, -126)
    scale = lax.bitcast_convert_type((kc + 127) << 23, jnp.float32)
    return p * scale


def _zero_rows(buf, nrows):
    @pl.loop(0, nrows)
    def _(j):
        for r in range(D // 16):
            buf[j, pl.ds(r * 16, 16)] = jnp.zeros((16,), jnp.float32)


def _zero_vec(buf, nelem):
    @pl.loop(0, nelem // 16)
    def _(j):
        buf[pl.ds(j * 16, 16)] = jnp.zeros((16,), jnp.float32)


def _zero_vec_i32(buf, nelem):
    @pl.loop(0, nelem // 16)
    def _(j):
        buf[pl.ds(j * 16, 16)] = jnp.zeros((16,), jnp.int32)


def _chunk_start(buf, acc_sh, tbl_hbm, elf_hbm, erf_hbm, src_hbm, dst_hbm,
                 base, hoff, emul, eoff):
    (srcv, dstv, ridx, lidx, didx, rows, elg, erg, eev, sm, scs) = buf
    pltpu.make_async_copy(rows, acc_sh.at[dstv], scs).wait()
    pltpu.sync_copy(src_hbm.at[pl.ds(base, CH)], srcv)
    pltpu.sync_copy(dst_hbm.at[pl.ds(base, CH)], dstv)
    for g in range(CH // 16):
        sl = pl.ds(g * 16, 16)
        sv = srcv[sl]
        dv = dstv[sl]
        ridx[sl] = sv + hoff
        lidx[sl] = sv * emul + eoff
        didx[sl] = dv * emul + eoff
    pltpu.async_copy(tbl_hbm.at[ridx], rows, sm)
    pltpu.async_copy(elf_hbm.at[lidx], elg, sm)
    pltpu.async_copy(erf_hbm.at[didx], erg, sm)


def _chunk_finish(buf, tbl_hbm, elf_hbm, erf_hbm, acc_sh, denp, cc, masks):
    (srcv, dstv, ridx, lidx, didx, rows, elg, erg, eev, sm, scs) = buf
    pltpu.make_async_copy(tbl_hbm.at[ridx], rows, sm).wait()
    pltpu.make_async_copy(elf_hbm.at[lidx], elg, sm).wait()
    pltpu.make_async_copy(erf_hbm.at[didx], erg, sm).wait()
    for g in range(CH // 16):
        sl = pl.ds(g * 16, 16)
        e = elg[sl] + erg[sl]
        e = jnp.where(e >= 0.0, e, e * 0.2)
        ee = _exp16(jnp.maximum(e - cc, -100.0))
        eev[sl] = ee
        dv = dstv[sl]
        for k in range(16):
            plsc.addupdate_scatter(denp, [dv], ee, mask=masks[k])

    @pl.loop(0, CH, unroll=4)
    def _(j):
        sc = eev[pl.ds(j, 16)][0]
        for r in range(D // 16):
            sl = pl.ds(r * 16, 16)
            rows[j, sl] = rows[j, sl] * sc

    pltpu.async_copy(rows, acc_sh.at[dstv], scs, add=True)


def _edge_pipeline(bufs, tbl_hbm, elf_hbm, erf_hbm, src_hbm, dst_hbm, ebase,
                   hoff, emul, eoff, acc_sh, denp, cc, masks):
    nch = NCHK
    start = lambda k, b: _chunk_start(bufs[b], acc_sh, tbl_hbm, elf_hbm,
                                      erf_hbm, src_hbm, dst_hbm,
                                      ebase + k * CH, hoff, emul, eoff)
    finish = lambda b: _chunk_finish(bufs[b], tbl_hbm, elf_hbm, erf_hbm,
                                     acc_sh, denp, cc, masks)
    for b in range(2):
        (srcv, dstv, _, _, _, rows, _, _, _, _, scs) = bufs[b]
        _zero_vec_i32(dstv, CH)
        _zero_rows(rows, CH)
        pltpu.async_copy(rows, acc_sh.at[dstv], scs, add=True)
    start(0, 0)
    if nch % 2 == 0:
        @pl.loop(0, nch - 2, step=2)
        def _(ci):
            for b in range(2):
                start(ci + b + 1, 1 - b)
                finish(b)

        start(nch - 1, 1)
        finish(0)
        finish(1)
    else:
        @pl.loop(0, nch - 1, step=2)
        def _(ci):
            for b in range(2):
                start(ci + b + 1, 1 - b)
                finish(b)

        finish(0)
    for b in range(2):
        (srcv, dstv, _, _, _, rows, _, _, _, _, scs) = bufs[b]
        pltpu.make_async_copy(rows, acc_sh.at[dstv], scs).wait()


def _reduce_divide_writeout(coff, s, acc_sh, den_sh, denp, dtmp, dsum, dbuf,
                            out_view):
    pltpu.sync_copy(denp, den_sh.at[pl.ds(coff + s * NPAD, NPAD)])
    plsc.subcore_barrier()
    _zero_vec(dsum, NPT)
    for k in range(NTILES):
        pltpu.sync_copy(den_sh.at[pl.ds(coff + k * NPAD + s * NPT, NPT)],
                        dtmp)

        @pl.loop(0, NPT // 16)
        def _(i):
            sl = pl.ds(i * 16, 16)
            dsum[sl] = dsum[sl] + dtmp[sl]

    @pl.loop(0, NPT // 16)
    def _(i):
        sl = pl.ds(i * 16, 16)
        dsum[sl] = 1.0 / (dsum[sl] + 1e-9)

    @pl.loop(0, NPT // ZCH)
    def _(k):
        pltpu.sync_copy(acc_sh.at[pl.ds(s * NPT + k * ZCH, ZCH)], dbuf)

        @pl.loop(0, ZCH)
        def _(j):
            dinv = dsum[pl.ds(k * ZCH + j, 16)][0]
            for r in range(D // 16):
                sl = pl.ds(r * 16, 16)
                dbuf[j, sl] = dbuf[j, sl] * dinv

        pltpu.sync_copy(dbuf, out_view.at[pl.ds(s * NPT + k * ZCH, ZCH)])


def _sc_l0_body(tbl_hbm, elf_hbm, erf_hbm, cel_hbm, cer_hbm, src_hbm, dst_hbm,
                out_hbm, den_sh, acc_sh, celv, cerv,
                srcv0, dstv0, ridx0, lidx0, didx0, rows0, elg0, erg0, eev0,
                sm0, scs0,
                srcv1, dstv1, ridx1, lidx1, didx1, rows1, elg1, erg1, eev1,
                sm1, scs1,
                denp, dtmp, dsum, dbuf):
    bufs = [(srcv0, dstv0, ridx0, lidx0, didx0, rows0, elg0, erg0, eev0,
             sm0, scs0),
            (srcv1, dstv1, ridx1, lidx1, didx1, rows1, elg1, erg1, eev1,
             sm1, scs1)]
    c = lax.axis_index("c")
    s = lax.axis_index("s")
    pltpu.sync_copy(cel_hbm, celv)
    pltpu.sync_copy(cer_hbm, cerv)
    ccv = celv[...] + cerv[...]
    lanes = lax.iota(jnp.int32, 16)
    masks = [lanes == k for k in range(16)]
    for hp in range(H0 // 2):
        h = c * (H0 // 2) + hp
        cc = jnp.where(c == 0, ccv[hp], ccv[H0 // 2 + hp])
        _zero_rows(dbuf, ZCH)

        @pl.loop(0, NPT // ZCH)
        def _(k):
            pltpu.sync_copy(dbuf, acc_sh.at[pl.ds(s * NPT + k * ZCH, ZCH)])
        _zero_vec(denp, NPAD)
        plsc.subcore_barrier()

        _edge_pipeline(bufs, tbl_hbm, elf_hbm, erf_hbm, src_hbm, dst_hbm,
                       s * EPT, h * N, H0, h, acc_sh, denp, cc, masks)

        plsc.subcore_barrier()
        _reduce_divide_writeout(c * (NTILES * NPAD), s, acc_sh, den_sh, denp,
                                dtmp, dsum, dbuf, out_hbm.at[h])
        plsc.subcore_barrier()


def _sc_l1_body(tbl_hbm, elf_hbm, erf_hbm, cel_hbm, cer_hbm, src_hbm, dst_hbm,
                out_hbm, den_sh, acc_sh, celv, cerv,
                srcv0, dstv0, ridx0, lidx0, didx0, rows0, elg0, erg0, eev0,
                sm0, scs0,
                srcv1, dstv1, ridx1, lidx1, didx1, rows1, elg1, erg1, eev1,
                sm1, scs1,
                denp, dtmp, dsum, dbuf):
    bufs = [(srcv0, dstv0, ridx0, lidx0, didx0, rows0, elg0, erg0, eev0,
             sm0, scs0),
            (srcv1, dstv1, ridx1, lidx1, didx1, rows1, elg1, erg1, eev1,
             sm1, scs1)]
    c = lax.axis_index("c")
    s = lax.axis_index("s")

    @pl.when(c == 0)
    def _():
        pltpu.sync_copy(cel_hbm, celv)
        pltpu.sync_copy(cer_hbm, cerv)
        cc = (celv[...] + cerv[...])[0]
        lanes = lax.iota(jnp.int32, 16)
        masks = [lanes == k for k in range(16)]
        _zero_rows(dbuf, ZCH)

        @pl.loop(0, NPT // ZCH)
        def _(k):
            pltpu.sync_copy(dbuf, acc_sh.at[pl.ds(s * NPT + k * ZCH, ZCH)])
        _zero_vec(denp, NPAD)
        plsc.subcore_barrier()

        _edge_pipeline(bufs, tbl_hbm, elf_hbm, erf_hbm, src_hbm, dst_hbm,
                       s * EPT, 0, 1, 0, acc_sh, denp, cc, masks)

        plsc.subcore_barrier()
        _reduce_divide_writeout(0, s, acc_sh, den_sh, denp, dtmp, dsum, dbuf,
                                out_hbm)


_SC_SCRATCH = [
    pltpu.VMEM_SHARED((NPAD, D), jnp.float32),
    pltpu.VMEM((16,), jnp.float32),
    pltpu.VMEM((16,), jnp.float32),
] + 2 * [
    pltpu.VMEM((CH,), jnp.int32),
    pltpu.VMEM((CH,), jnp.int32),
    pltpu.VMEM((CH,), jnp.int32),
    pltpu.VMEM((CH,), jnp.int32),
    pltpu.VMEM((CH,), jnp.int32),
    pltpu.VMEM((CH, D), jnp.float32),
    pltpu.VMEM((CH,), jnp.float32),
    pltpu.VMEM((CH,), jnp.float32),
    pltpu.VMEM((CH + 16,), jnp.float32),
    pltpu.SemaphoreType.DMA,
    pltpu.SemaphoreType.DMA,
] + [
    pltpu.VMEM((NPAD,), jnp.float32),
    pltpu.VMEM((NPT,), jnp.float32),
    pltpu.VMEM((NPT + 16,), jnp.float32),
    pltpu.VMEM((ZCH, D), jnp.float32),
]


@functools.lru_cache(maxsize=None)
def _sc_l0_kernel():
    return pl.kernel(
        _sc_l0_body,
        out_type=(jax.ShapeDtypeStruct((H0, NPAD, D), jnp.float32),
                  jax.ShapeDtypeStruct((2 * NTILES * NPAD,), jnp.float32)),
        mesh=_sc_mesh(),
        scratch_types=list(_SC_SCRATCH),
        compiler_params=pltpu.CompilerParams(needs_layout_passes=False),
    )


def _sc_l0(*args):
    return _sc_l0_kernel()(*args)[0]


@functools.lru_cache(maxsize=None)
def _sc_l1_kernel():
    return pl.kernel(
        _sc_l1_body,
        out_type=(jax.ShapeDtypeStruct((NPAD, D), jnp.float32),
                  jax.ShapeDtypeStruct((2 * NTILES * NPAD,), jnp.float32)),
        mesh=_sc_mesh(),
        scratch_types=list(_SC_SCRATCH),
        compiler_params=pltpu.CompilerParams(needs_layout_passes=False),
    )


def _sc_l1(*args):
    return _sc_l1_kernel()(*args)[0]


def _tc2_body(acc_ref, res_ref, b0_ref, fw1_ref, al1_ref, ar1_ref,
              h1_ref, tbl1_ref, el1_ref, er1_ref, cs1_ref, cel_ref, cer_ref):
    i = pl.program_id(0)
    acc = 0.0
    for k in range(H0):
        gat = (acc_ref[k] + res_ref[k]
               + b0_ref[:, k * D:(k + 1) * D])
        mu = jnp.mean(gat, axis=-1, keepdims=True)
        var = jnp.mean((gat - mu) ** 2, axis=-1, keepdims=True)
        acc = acc + (gat - mu) / jnp.sqrt(var + 1e-5)
    h1 = acc * (1.0 / H0)
    h1_ref[...] = h1
    feat1 = jnp.dot(h1, fw1_ref[...], preferred_element_type=jnp.float32)
    el1 = jnp.sum(feat1 * al1_ref[...], axis=-1, keepdims=True)
    er1 = jnp.sum(feat1 * ar1_ref[...], axis=-1, keepdims=True)
    el1_ref[...] = el1
    er1_ref[...] = er1
    tbl1_ref[...] = feat1

    @pl.when(i == 0)
    def _():
        cs1_ref[...] = jnp.zeros_like(cs1_ref)
        cel_ref[...] = jnp.full_like(cel_ref, NEG)
        cer_ref[...] = jnp.full_like(cer_ref, NEG)

    cs1_ref[...] += jnp.sum(h1, axis=0, keepdims=True)
    cel_ref[...] = jnp.maximum(cel_ref[...],
                               jnp.full((1, 16), jnp.max(el1), jnp.float32))
    cer_ref[...] = jnp.maximum(cer_ref[...],
                               jnp.full((1, 16), jnp.max(er1), jnp.float32))


def _tc2(acc0, res0, bias0, fc1_W, attn_l1, attn_r1):
    rep = lambda shape: pl.BlockSpec(shape, lambda i: tuple(0 for _ in shape))
    return pl.pallas_call(
        _tc2_body,
        grid=(GRID,),
        in_specs=[
            pl.BlockSpec((H0, NB, D), lambda i: (0, i, 0)),
            pl.BlockSpec((H0, NB, D), lambda i: (0, i, 0)),
            rep((1, H0 * D)), rep((D, D)), rep((1, D)), rep((1, D)),
        ],
        out_specs=[
            pl.BlockSpec((NB, D), lambda i: (i, 0)),
            pl.BlockSpec((NB, D), lambda i: (i, 0)),
            pl.BlockSpec((NB, 1), lambda i: (i, 0)),
            pl.BlockSpec((NB, 1), lambda i: (i, 0)),
            rep((1, D)), rep((1, 16)), rep((1, 16)),
        ],
        out_shape=[
            jax.ShapeDtypeStruct((N, D), jnp.float32),
            jax.ShapeDtypeStruct((N, D), jnp.float32),
            jax.ShapeDtypeStruct((N, 1), jnp.float32),
            jax.ShapeDtypeStruct((N, 1), jnp.float32),
            jax.ShapeDtypeStruct((1, D), jnp.float32),
            jax.ShapeDtypeStruct((1, 16), jnp.float32),
            jax.ShapeDtypeStruct((1, 16), jnp.float32),
        ],
    )(acc0, res0, bias0, fc1_W, attn_l1, attn_r1)


def _tc3_body(acc_ref, h1_ref, b1_ref, hg0_ref, cs1_ref, gl0w_ref, gl0b_ref,
              gl1w_ref, gl1b_ref, m0w_ref, m0b_ref, m1w_ref, m1b_ref,
              m2w_ref, m2b_ref, out_ref, cs2_ref):
    i = pl.program_id(0)
    gat = acc_ref[...] + h1_ref[...] + b1_ref[...]
    mu = jnp.mean(gat, axis=-1, keepdims=True)
    var = jnp.mean((gat - mu) ** 2, axis=-1, keepdims=True)
    h2 = (gat - mu) / jnp.sqrt(var + 1e-5)

    @pl.when(i == 0)
    def _():
        cs2_ref[...] = jnp.zeros_like(cs2_ref)

    cs2_ref[...] += jnp.sum(h2, axis=0, keepdims=True)

    @pl.when(i == GRID - 1)
    def _():
        dot = lambda a, b: jnp.dot(a, b, preferred_element_type=jnp.float32)
        hg = (hg0_ref[...]
              + _lrelu(dot(cs1_ref[...], gl0w_ref[...]) + gl0b_ref[...], 0.01)
              + _lrelu(dot(cs2_ref[...], gl1w_ref[...]) + gl1b_ref[...], 0.01))
        hg = dot(hg, m0w_ref[...]) + m0b_ref[...]
        hg = dot(jnp.maximum(hg, 0.0), m1w_ref[...]) + m1b_ref[...]
        hg = dot(jnp.maximum(hg, 0.0), m2w_ref[...]) + m2b_ref[...]
        out_ref[...] = hg


def _tc3(acc1, h1, bias1, hg0, cs1, gl0_W, gl0_b, gl1_W, gl1_b,
         m0_W, m0_b, m1_W, m1_b, m2_W, m2_b):
    rep = lambda shape: pl.BlockSpec(shape, lambda i: tuple(0 for _ in shape))
    mlp = m0_W.shape[1]
    return pl.pallas_call(
        _tc3_body,
        grid=(GRID,),
        in_specs=[
            pl.BlockSpec((NB, D), lambda i: (i, 0)),
            pl.BlockSpec((NB, D), lambda i: (i, 0)),
            rep((1, D)), rep((1, D)), rep((1, D)),
            rep((D, D)), rep((1, D)), rep((D, D)), rep((1, D)),
            rep((D, mlp)), rep((1, mlp)), rep((mlp, mlp)), rep((1, mlp)),
            rep((mlp, mlp)), rep((1, mlp)),
        ],
        out_specs=[rep((1, mlp))],
        out_shape=[jax.ShapeDtypeStruct((1, mlp), jnp.float32)],
        scratch_shapes=[pltpu.VMEM((1, D), jnp.float32)],
    )(acc1, h1, bias1, hg0, cs1, gl0_W, gl0_b, gl1_W, gl1_b,
      m0_W, m0_b, m1_W, m1_b, m2_W, m2_b)[0]


@jax.jit
def kernel(node_features, edge_index, proj_W, proj_b, fc0_W, attn_l0, attn_r0,
           res0_W, bias0, gl0_W, gl0_b, fc1_W, attn_l1, attn_r1, bias1,
           gl1_W, gl1_b, m0_W, m0_b, m1_W, m1_b, m2_W, m2_b):
    pad = EPAD - E
    src = jnp.concatenate([edge_index[0], jnp.zeros(pad, jnp.int32)])
    dst = jnp.concatenate([edge_index[1], jnp.full(pad, N, jnp.int32)])
    erpad = jnp.full(16, NEG, jnp.float32)
    row = lambda v: v.reshape(1, -1)

    tbl0, elf0, erf0, res0, hg0, cel0, cer0 = _tc1(
        node_features, proj_W, row(proj_b), fc0_W,
        row(attn_l0.reshape(-1)), row(attn_r0.reshape(-1)), res0_W)

    acc0 = _sc_l0(tbl0.reshape(H0 * N, D), elf0.reshape(-1),
                  jnp.concatenate([erf0.reshape(-1), erpad]),
                  cel0.reshape(-1), cer0.reshape(-1), src, dst)

    h1, tbl1, el1, er1, cs1, cel1, cer1 = _tc2(
        acc0.reshape(H0, NPAD, D), res0, row(bias0), fc1_W,
        row(attn_l1.reshape(-1)), row(attn_r1.reshape(-1)))

    acc1 = _sc_l1(tbl1, el1.reshape(-1),
                  jnp.concatenate([er1.reshape(-1), erpad]),
                  cel1.reshape(-1), cer1.reshape(-1), src, dst)

    return _tc3(acc1, h1, row(bias1), hg0, cs1, gl0_W, row(gl0_b),
                gl1_W, row(gl1_b), m0_W, row(m0_b), m1_W, row(m1_b),
                m2_W, row(m2_b))


# speedup vs baseline: 1.2100x; 1.2100x over previous
import functools

import jax
import jax.numpy as jnp
from jax import lax
from jax.experimental import pallas as pl
from jax.experimental.pallas import tpu as pltpu
from jax.experimental.pallas import tpu_sc as plsc

N = 10000
E = 320000
D = 128
H0 = 8
NB = 400
GRID = N // NB
NTILES = 16
NPAD = 10240
NPT = NPAD // NTILES
ZCH = 128
CH = 64
EPAD = 320512
EPT = EPAD // NTILES
NCHK = EPT // CH
NEG = -1e30


def _lrelu(x, slope):
    return jnp.where(x >= 0.0, x, x * slope)


def _tc1_body(x_ref, pw_ref, pb_ref, fw_ref, al_ref, ar_ref, rw_ref,
              tbl_ref, elf_ref, erf_ref, res_ref, hg_ref, cel_ref, cer_ref):
    i = pl.program_id(0)
    x = x_ref[...]
    h = jnp.dot(x, pw_ref[...], preferred_element_type=jnp.float32) + pb_ref[...]
    els = []
    ers = []
    for k in range(H0):
        fk = jnp.dot(h, fw_ref[:, k * D:(k + 1) * D],
                     preferred_element_type=jnp.float32)
        tbl_ref[k] = fk
        res_ref[k] = jnp.dot(h, rw_ref[:, k * D:(k + 1) * D],
                             preferred_element_type=jnp.float32)
        els.append(jnp.sum(fk * al_ref[:, k * D:(k + 1) * D], axis=-1,
                           keepdims=True))
        ers.append(jnp.sum(fk * ar_ref[:, k * D:(k + 1) * D], axis=-1,
                           keepdims=True))
    el = jnp.concatenate(els, axis=-1)
    er = jnp.concatenate(ers, axis=-1)
    elf_ref[...] = el
    erf_ref[...] = er

    @pl.when(i == 0)
    def _():
        hg_ref[...] = jnp.zeros_like(hg_ref)
        cel_ref[...] = jnp.full_like(cel_ref, NEG)
        cer_ref[...] = jnp.full_like(cer_ref, NEG)

    hg_ref[...] += jnp.sum(h, axis=0, keepdims=True)
    zpad = jnp.full((1, 8), NEG, jnp.float32)
    melp = jnp.concatenate([jnp.max(el, axis=0, keepdims=True), zpad], axis=-1)
    merp = jnp.concatenate([jnp.max(er, axis=0, keepdims=True), zpad], axis=-1)
    cel_ref[...] = jnp.maximum(cel_ref[...], melp)
    cer_ref[...] = jnp.maximum(cer_ref[...], merp)


def _tc1(x, proj_W, proj_b, fc0_W, attn_l0, attn_r0, res0_W):
    rep = lambda shape: pl.BlockSpec(shape, lambda i: tuple(0 for _ in shape))
    return pl.pallas_call(
        _tc1_body,
        grid=(GRID,),
        in_specs=[
            pl.BlockSpec((NB, D), lambda i: (i, 0)),
            rep((D, D)), rep((1, D)), rep((D, H0 * D)),
            rep((1, H0 * D)), rep((1, H0 * D)), rep((D, H0 * D)),
        ],
        out_specs=[
            pl.BlockSpec((H0, NB, D), lambda i: (0, i, 0)),
            pl.BlockSpec((NB, H0), lambda i: (i, 0)),
            pl.BlockSpec((NB, H0), lambda i: (i, 0)),
            pl.BlockSpec((H0, NB, D), lambda i: (0, i, 0)),
            rep((1, D)), rep((1, 16)), rep((1, 16)),
        ],
        out_shape=[
            jax.ShapeDtypeStruct((H0, N, D), jnp.float32),
            jax.ShapeDtypeStruct((N, H0), jnp.float32),
            jax.ShapeDtypeStruct((N, H0), jnp.float32),
            jax.ShapeDtypeStruct((H0, N, D), jnp.float32),
            jax.ShapeDtypeStruct((1, D), jnp.float32),
            jax.ShapeDtypeStruct((1, 16), jnp.float32),
            jax.ShapeDtypeStruct((1, 16), jnp.float32),
        ],
    )(x, proj_W, proj_b, fc0_W, attn_l0, attn_r0, res0_W)


@functools.lru_cache(maxsize=None)
def _sc_mesh():
    return plsc.VectorSubcoreMesh(core_axis_name="c", subcore_axis_name="s")


def _exp16(x):
    y = x * 1.4426950408889634
    k = y.astype(jnp.int32)
    k = jnp.where(y < k.astype(jnp.float32), k - 1, k)
    f = y - k.astype(jnp.float32)
    p = jnp.full((16,), 1.8775767e-3, jnp.float32)
    for coef in (8.9893397e-3, 5.5826318e-2, 2.4015361e-1, 6.9315308e-1, 1.0):
        p = p * f + coef
    kc = jnp.maximum(k, -126)
    scale = lax.bitcast_convert_type((kc + 127) << 23, jnp.float32)
    return p * scale


def _zero_rows(buf, nrows):
    @pl.loop(0, nrows)
    def _(j):
        for r in range(D // 16):
            buf[j, pl.ds(r * 16, 16)] = jnp.zeros((16,), jnp.float32)


def _zero_vec(buf, nelem):
    @pl.loop(0, nelem // 16)
    def _(j):
        buf[pl.ds(j * 16, 16)] = jnp.zeros((16,), jnp.float32)


def _zero_vec_i32(buf, nelem):
    @pl.loop(0, nelem // 16)
    def _(j):
        buf[pl.ds(j * 16, 16)] = jnp.zeros((16,), jnp.int32)


def _chunk_start(buf, acc_sh, tbl_hbm, elf_hbm, erf_hbm, src_hbm, dst_hbm,
                 base, hoff, emul, eoff):
    (srcv, dstv, ridx, lidx, didx, rows, elg, erg, eev, sm, scs) = buf
    pltpu.make_async_copy(rows, acc_sh.at[dstv], scs).wait()
    pltpu.async_copy(src_hbm.at[pl.ds(base, CH)], srcv, sm)
    pltpu.async_copy(dst_hbm.at[pl.ds(base, CH)], dstv, sm)
    pltpu.make_async_copy(src_hbm.at[pl.ds(base, CH)], srcv, sm).wait()
    pltpu.make_async_copy(dst_hbm.at[pl.ds(base, CH)], dstv, sm).wait()
    for g in range(CH // 16):
        sl = pl.ds(g * 16, 16)
        sv = srcv[sl]
        dv = dstv[sl]
        ridx[sl] = sv + hoff
        lidx[sl] = sv * emul + eoff
        didx[sl] = dv * emul + eoff
    pltpu.async_copy(tbl_hbm.at[ridx], rows, sm)
    pltpu.async_copy(elf_hbm.at[lidx], elg, sm)
    pltpu.async_copy(erf_hbm.at[didx], erg, sm)


def _chunk_finish(buf, tbl_hbm, elf_hbm, erf_hbm, acc_sh, denp, cc, masks):
    (srcv, dstv, ridx, lidx, didx, rows, elg, erg, eev, sm, scs) = buf
    pltpu.make_async_copy(tbl_hbm.at[ridx], rows, sm).wait()
    pltpu.make_async_copy(elf_hbm.at[lidx], elg, sm).wait()
    pltpu.make_async_copy(erf_hbm.at[didx], erg, sm).wait()
    for g in range(CH // 16):
        sl = pl.ds(g * 16, 16)
        e = elg[sl] + erg[sl]
        e = jnp.where(e >= 0.0, e, e * 0.2)
        ee = _exp16(jnp.maximum(e - cc, -100.0))
        eev[sl] = ee
        dv = dstv[sl]
        for k in range(16):
            plsc.addupdate_scatter(denp, [dv], ee, mask=masks[k])

    @pl.loop(0, CH, unroll=4)
    def _(j):
        sc = eev[pl.ds(j, 16)][0]
        for r in range(D // 16):
            sl = pl.ds(r * 16, 16)
            rows[j, sl] = rows[j, sl] * sc

    pltpu.async_copy(rows, acc_sh.at[dstv], scs, add=True)


def _edge_pipeline(bufs, tbl_hbm, elf_hbm, erf_hbm, src_hbm, dst_hbm, ebase,
                   hoff, emul, eoff, acc_sh, denp, cc, masks):
    nch = NCHK
    start = lambda k, b: _chunk_start(bufs[b], acc_sh, tbl_hbm, elf_hbm,
                                      erf_hbm, src_hbm, dst_hbm,
                                      ebase + k * CH, hoff, emul, eoff)
    finish = lambda b: _chunk_finish(bufs[b], tbl_hbm, elf_hbm, erf_hbm,
                                     acc_sh, denp, cc, masks)
    for b in range(2):
        (srcv, dstv, _, _, _, rows, _, _, _, _, scs) = bufs[b]
        _zero_vec_i32(dstv, CH)
        _zero_rows(rows, CH)
        pltpu.async_copy(rows, acc_sh.at[dstv], scs, add=True)
    start(0, 0)
    if nch % 2 == 0:
        @pl.loop(0, nch - 2, step=2)
        def _(ci):
            for b in range(2):
                start(ci + b + 1, 1 - b)
                finish(b)

        start(nch - 1, 1)
        finish(0)
        finish(1)
    else:
        @pl.loop(0, nch - 1, step=2)
        def _(ci):
            for b in range(2):
                start(ci + b + 1, 1 - b)
                finish(b)

        finish(0)
    for b in range(2):
        (srcv, dstv, _, _, _, rows, _, _, _, _, scs) = bufs[b]
        pltpu.make_async_copy(rows, acc_sh.at[dstv], scs).wait()


def _reduce_divide_writeout(coff, s, acc_sh, den_sh, denp, dtmp, dsum, dbuf,
                            out_view):
    pltpu.sync_copy(denp, den_sh.at[pl.ds(coff + s * NPAD, NPAD)])
    plsc.subcore_barrier()
    _zero_vec(dsum, NPT)
    for k in range(NTILES):
        pltpu.sync_copy(den_sh.at[pl.ds(coff + k * NPAD + s * NPT, NPT)],
                        dtmp)

        @pl.loop(0, NPT // 16)
        def _(i):
            sl = pl.ds(i * 16, 16)
            dsum[sl] = dsum[sl] + dtmp[sl]

    @pl.loop(0, NPT // 16)
    def _(i):
        sl = pl.ds(i * 16, 16)
        dsum[sl] = 1.0 / (dsum[sl] + 1e-9)

    @pl.loop(0, NPT // ZCH)
    def _(k):
        pltpu.sync_copy(acc_sh.at[pl.ds(s * NPT + k * ZCH, ZCH)], dbuf)

        @pl.loop(0, ZCH)
        def _(j):
            dinv = dsum[pl.ds(k * ZCH + j, 16)][0]
            for r in range(D // 16):
                sl = pl.ds(r * 16, 16)
                dbuf[j, sl] = dbuf[j, sl] * dinv

        pltpu.sync_copy(dbuf, out_view.at[pl.ds(s * NPT + k * ZCH, ZCH)])


def _sc_l0_body(tbl_hbm, elf_hbm, erf_hbm, cel_hbm, cer_hbm, src_hbm, dst_hbm,
                out_hbm, den_sh, acc_sh, celv, cerv,
                srcv0, dstv0, ridx0, lidx0, didx0, rows0, elg0, erg0, eev0,
                sm0, scs0,
                srcv1, dstv1, ridx1, lidx1, didx1, rows1, elg1, erg1, eev1,
                sm1, scs1,
                denp, dtmp, dsum, dbuf):
    bufs = [(srcv0, dstv0, ridx0, lidx0, didx0, rows0, elg0, erg0, eev0,
             sm0, scs0),
            (srcv1, dstv1, ridx1, lidx1, didx1, rows1, elg1, erg1, eev1,
             sm1, scs1)]
    c = lax.axis_index("c")
    s = lax.axis_index("s")
    pltpu.sync_copy(cel_hbm, celv)
    pltpu.sync_copy(cer_hbm, cerv)
    ccv = celv[...] + cerv[...]
    lanes = lax.iota(jnp.int32, 16)
    masks = [lanes == k for k in range(16)]
    for hp in range(H0 // 2):
        h = c * (H0 // 2) + hp
        cc = jnp.where(c == 0, ccv[hp], ccv[H0 // 2 + hp])
        _zero_rows(dbuf, ZCH)

        @pl.loop(0, NPT // ZCH)
        def _(k):
            pltpu.sync_copy(dbuf, acc_sh.at[pl.ds(s * NPT + k * ZCH, ZCH)])
        _zero_vec(denp, NPAD)
        plsc.subcore_barrier()

        _edge_pipeline(bufs, tbl_hbm, elf_hbm, erf_hbm, src_hbm, dst_hbm,
                       s * EPT, h * N, H0, h, acc_sh, denp, cc, masks)

        plsc.subcore_barrier()
        _reduce_divide_writeout(c * (NTILES * NPAD), s, acc_sh, den_sh, denp,
                                dtmp, dsum, dbuf, out_hbm.at[h])
        plsc.subcore_barrier()


def _sc_l1_body(tbl_hbm, elf_hbm, erf_hbm, cel_hbm, cer_hbm, src_hbm, dst_hbm,
                out_hbm, den_sh, acc_sh, celv, cerv,
                srcv0, dstv0, ridx0, lidx0, didx0, rows0, elg0, erg0, eev0,
                sm0, scs0,
                srcv1, dstv1, ridx1, lidx1, didx1, rows1, elg1, erg1, eev1,
                sm1, scs1,
                denp, dtmp, dsum, dbuf):
    bufs = [(srcv0, dstv0, ridx0, lidx0, didx0, rows0, elg0, erg0, eev0,
             sm0, scs0),
            (srcv1, dstv1, ridx1, lidx1, didx1, rows1, elg1, erg1, eev1,
             sm1, scs1)]
    c = lax.axis_index("c")
    s = lax.axis_index("s")

    @pl.when(c == 0)
    def _():
        pltpu.sync_copy(cel_hbm, celv)
        pltpu.sync_copy(cer_hbm, cerv)
        cc = (celv[...] + cerv[...])[0]
        lanes = lax.iota(jnp.int32, 16)
        masks = [lanes == k for k in range(16)]
        _zero_rows(dbuf, ZCH)

        @pl.loop(0, NPT // ZCH)
        def _(k):
            pltpu.sync_copy(dbuf, acc_sh.at[pl.ds(s * NPT + k * ZCH, ZCH)])
        _zero_vec(denp, NPAD)
        plsc.subcore_barrier()

        _edge_pipeline(bufs, tbl_hbm, elf_hbm, erf_hbm, src_hbm, dst_hbm,
                       s * EPT, 0, 1, 0, acc_sh, denp, cc, masks)

        plsc.subcore_barrier()
        _reduce_divide_writeout(0, s, acc_sh, den_sh, denp, dtmp, dsum, dbuf,
                                out_hbm)


_SC_SCRATCH = [
    pltpu.VMEM_SHARED((NPAD, D), jnp.float32),
    pltpu.VMEM((16,), jnp.float32),
    pltpu.VMEM((16,), jnp.float32),
] + 2 * [
    pltpu.VMEM((CH,), jnp.int32),
    pltpu.VMEM((CH,), jnp.int32),
    pltpu.VMEM((CH,), jnp.int32),
    pltpu.VMEM((CH,), jnp.int32),
    pltpu.VMEM((CH,), jnp.int32),
    pltpu.VMEM((CH, D), jnp.float32),
    pltpu.VMEM((CH,), jnp.float32),
    pltpu.VMEM((CH,), jnp.float32),
    pltpu.VMEM((CH + 16,), jnp.float32),
    pltpu.SemaphoreType.DMA,
    pltpu.SemaphoreType.DMA,
] + [
    pltpu.VMEM((NPAD,), jnp.float32),
    pltpu.VMEM((NPT,), jnp.float32),
    pltpu.VMEM((NPT + 16,), jnp.float32),
    pltpu.VMEM((ZCH, D), jnp.float32),
]


@functools.lru_cache(maxsize=None)
def _sc_l0_kernel():
    return pl.kernel(
        _sc_l0_body,
        out_type=(jax.ShapeDtypeStruct((H0, NPAD, D), jnp.float32),
                  jax.ShapeDtypeStruct((2 * NTILES * NPAD,), jnp.float32)),
        mesh=_sc_mesh(),
        scratch_types=list(_SC_SCRATCH),
        compiler_params=pltpu.CompilerParams(needs_layout_passes=False),
    )


def _sc_l0(*args):
    return _sc_l0_kernel()(*args)[0]


@functools.lru_cache(maxsize=None)
def _sc_l1_kernel():
    return pl.kernel(
        _sc_l1_body,
        out_type=(jax.ShapeDtypeStruct((NPAD, D), jnp.float32),
                  jax.ShapeDtypeStruct((2 * NTILES * NPAD,), jnp.float32)),
        mesh=_sc_mesh(),
        scratch_types=list(_SC_SCRATCH),
        compiler_params=pltpu.CompilerParams(needs_layout_passes=False),
    )


def _sc_l1(*args):
    return _sc_l1_kernel()(*args)[0]


def _tc2_body(acc_ref, res_ref, b0_ref, fw1_ref, al1_ref, ar1_ref,
              h1_ref, tbl1_ref, el1_ref, er1_ref, cs1_ref, cel_ref, cer_ref):
    i = pl.program_id(0)
    acc = 0.0
    for k in range(H0):
        gat = (acc_ref[k] + res_ref[k]
               + b0_ref[:, k * D:(k + 1) * D])
        mu = jnp.mean(gat, axis=-1, keepdims=True)
        var = jnp.mean((gat - mu) ** 2, axis=-1, keepdims=True)
        acc = acc + (gat - mu) / jnp.sqrt(var + 1e-5)
    h1 = acc * (1.0 / H0)
    h1_ref[...] = h1
    feat1 = jnp.dot(h1, fw1_ref[...], preferred_element_type=jnp.float32)
    el1 = jnp.sum(feat1 * al1_ref[...], axis=-1, keepdims=True)
    er1 = jnp.sum(feat1 * ar1_ref[...], axis=-1, keepdims=True)
    el1_ref[...] = el1
    er1_ref[...] = er1
    tbl1_ref[...] = feat1

    @pl.when(i == 0)
    def _():
        cs1_ref[...] = jnp.zeros_like(cs1_ref)
        cel_ref[...] = jnp.full_like(cel_ref, NEG)
        cer_ref[...] = jnp.full_like(cer_ref, NEG)

    cs1_ref[...] += jnp.sum(h1, axis=0, keepdims=True)
    cel_ref[...] = jnp.maximum(cel_ref[...],
                               jnp.full((1, 16), jnp.max(el1), jnp.float32))
    cer_ref[...] = jnp.maximum(cer_ref[...],
                               jnp.full((1, 16), jnp.max(er1), jnp.float32))


def _tc2(acc0, res0, bias0, fc1_W, attn_l1, attn_r1):
    rep = lambda shape: pl.BlockSpec(shape, lambda i: tuple(0 for _ in shape))
    return pl.pallas_call(
        _tc2_body,
        grid=(GRID,),
        in_specs=[
            pl.BlockSpec((H0, NB, D), lambda i: (0, i, 0)),
            pl.BlockSpec((H0, NB, D), lambda i: (0, i, 0)),
            rep((1, H0 * D)), rep((D, D)), rep((1, D)), rep((1, D)),
        ],
        out_specs=[
            pl.BlockSpec((NB, D), lambda i: (i, 0)),
            pl.BlockSpec((NB, D), lambda i: (i, 0)),
            pl.BlockSpec((NB, 1), lambda i: (i, 0)),
            pl.BlockSpec((NB, 1), lambda i: (i, 0)),
            rep((1, D)), rep((1, 16)), rep((1, 16)),
        ],
        out_shape=[
            jax.ShapeDtypeStruct((N, D), jnp.float32),
            jax.ShapeDtypeStruct((N, D), jnp.float32),
            jax.ShapeDtypeStruct((N, 1), jnp.float32),
            jax.ShapeDtypeStruct((N, 1), jnp.float32),
            jax.ShapeDtypeStruct((1, D), jnp.float32),
            jax.ShapeDtypeStruct((1, 16), jnp.float32),
            jax.ShapeDtypeStruct((1, 16), jnp.float32),
        ],
    )(acc0, res0, bias0, fc1_W, attn_l1, attn_r1)


def _tc3_body(acc_ref, h1_ref, b1_ref, hg0_ref, cs1_ref, gl0w_ref, gl0b_ref,
              gl1w_ref, gl1b_ref, m0w_ref, m0b_ref, m1w_ref, m1b_ref,
              m2w_ref, m2b_ref, out_ref, cs2_ref):
    i = pl.program_id(0)
    gat = acc_ref[...] + h1_ref[...] + b1_ref[...]
    mu = jnp.mean(gat, axis=-1, keepdims=True)
    var = jnp.mean((gat - mu) ** 2, axis=-1, keepdims=True)
    h2 = (gat - mu) / jnp.sqrt(var + 1e-5)

    @pl.when(i == 0)
    def _():
        cs2_ref[...] = jnp.zeros_like(cs2_ref)

    cs2_ref[...] += jnp.sum(h2, axis=0, keepdims=True)

    @pl.when(i == GRID - 1)
    def _():
        dot = lambda a, b: jnp.dot(a, b, preferred_element_type=jnp.float32)
        hg = (hg0_ref[...]
              + _lrelu(dot(cs1_ref[...], gl0w_ref[...]) + gl0b_ref[...], 0.01)
              + _lrelu(dot(cs2_ref[...], gl1w_ref[...]) + gl1b_ref[...], 0.01))
        hg = dot(hg, m0w_ref[...]) + m0b_ref[...]
        hg = dot(jnp.maximum(hg, 0.0), m1w_ref[...]) + m1b_ref[...]
        hg = dot(jnp.maximum(hg, 0.0), m2w_ref[...]) + m2b_ref[...]
        out_ref[...] = hg


def _tc3(acc1, h1, bias1, hg0, cs1, gl0_W, gl0_b, gl1_W, gl1_b,
         m0_W, m0_b, m1_W, m1_b, m2_W, m2_b):
    rep = lambda shape: pl.BlockSpec(shape, lambda i: tuple(0 for _ in shape))
    mlp = m0_W.shape[1]
    return pl.pallas_call(
        _tc3_body,
        grid=(GRID,),
        in_specs=[
            pl.BlockSpec((NB, D), lambda i: (i, 0)),
            pl.BlockSpec((NB, D), lambda i: (i, 0)),
            rep((1, D)), rep((1, D)), rep((1, D)),
            rep((D, D)), rep((1, D)), rep((D, D)), rep((1, D)),
            rep((D, mlp)), rep((1, mlp)), rep((mlp, mlp)), rep((1, mlp)),
            rep((mlp, mlp)), rep((1, mlp)),
        ],
        out_specs=[rep((1, mlp))],
        out_shape=[jax.ShapeDtypeStruct((1, mlp), jnp.float32)],
        scratch_shapes=[pltpu.VMEM((1, D), jnp.float32)],
    )(acc1, h1, bias1, hg0, cs1, gl0_W, gl0_b, gl1_W, gl1_b,
      m0_W, m0_b, m1_W, m1_b, m2_W, m2_b)[0]


@jax.jit
def kernel(node_features, edge_index, proj_W, proj_b, fc0_W, attn_l0, attn_r0,
           res0_W, bias0, gl0_W, gl0_b, fc1_W, attn_l1, attn_r1, bias1,
           gl1_W, gl1_b, m0_W, m0_b, m1_W, m1_b, m2_W, m2_b):
    pad = EPAD - E
    src = jnp.concatenate([edge_index[0], jnp.zeros(pad, jnp.int32)])
    dst = jnp.concatenate([edge_index[1], jnp.full(pad, N, jnp.int32)])
    erpad = jnp.full(16, NEG, jnp.float32)
    row = lambda v: v.reshape(1, -1)

    tbl0, elf0, erf0, res0, hg0, cel0, cer0 = _tc1(
        node_features, proj_W, row(proj_b), fc0_W,
        row(attn_l0.reshape(-1)), row(attn_r0.reshape(-1)), res0_W)

    acc0 = _sc_l0(tbl0.reshape(H0 * N, D), elf0.reshape(-1),
                  jnp.concatenate([erf0.reshape(-1), erpad]),
                  cel0.reshape(-1), cer0.reshape(-1), src, dst)

    h1, tbl1, el1, er1, cs1, cel1, cer1 = _tc2(
        acc0.reshape(H0, NPAD, D), res0, row(bias0), fc1_W,
        row(attn_l1.reshape(-1)), row(attn_r1.reshape(-1)))

    acc1 = _sc_l1(tbl1, el1.reshape(-1),
                  jnp.concatenate([er1.reshape(-1), erpad]),
                  cel1.reshape(-1), cer1.reshape(-1), src, dst)

    return _tc3(acc1, h1, row(bias1), hg0, cs1, gl0_W, row(gl0_b),
                gl1_W, row(gl1_b), m0_W, row(m0_b), m1_W, row(m1_b),
                m2_W, row(m2_b))


# speedup vs baseline: 1.5510x; 1.2818x over previous
import functools

import jax
import jax.numpy as jnp
from jax import lax
from jax.experimental import pallas as pl
from jax.experimental.pallas import tpu as pltpu
from jax.experimental.pallas import tpu_sc as plsc

N = 10000
E = 320000
D = 128
H0 = 8
NB = 400
GRID = N // NB
NTILES = 16
NPAD = 10240
NPT = NPAD // NTILES
ZCH = 128
CH = 64
EPAD = 320512
EPT = EPAD // NTILES
NCHK = EPT // CH
EXTRA = 2 * CH
NEG = -1e30


def _lrelu(x, slope):
    return jnp.where(x >= 0.0, x, x * slope)


def _tc1_body(x_ref, pw_ref, pb_ref, fw_ref, al_ref, ar_ref, rw_ref,
              tbl_ref, elf_ref, erf_ref, res_ref, hg_ref, cel_ref, cer_ref):
    i = pl.program_id(0)
    x = x_ref[...]
    h = jnp.dot(x, pw_ref[...], preferred_element_type=jnp.float32) + pb_ref[...]
    els = []
    ers = []
    for k in range(H0):
        fk = jnp.dot(h, fw_ref[:, k * D:(k + 1) * D],
                     preferred_element_type=jnp.float32)
        tbl_ref[k] = fk
        res_ref[k] = jnp.dot(h, rw_ref[:, k * D:(k + 1) * D],
                             preferred_element_type=jnp.float32)
        els.append(jnp.sum(fk * al_ref[:, k * D:(k + 1) * D], axis=-1,
                           keepdims=True))
        ers.append(jnp.sum(fk * ar_ref[:, k * D:(k + 1) * D], axis=-1,
                           keepdims=True))
    el = jnp.concatenate(els, axis=-1)
    er = jnp.concatenate(ers, axis=-1)
    elf_ref[...] = el
    erf_ref[...] = er

    @pl.when(i == 0)
    def _():
        hg_ref[...] = jnp.zeros_like(hg_ref)
        cel_ref[...] = jnp.full_like(cel_ref, NEG)
        cer_ref[...] = jnp.full_like(cer_ref, NEG)

    hg_ref[...] += jnp.sum(h, axis=0, keepdims=True)
    zpad = jnp.full((1, 8), NEG, jnp.float32)
    melp = jnp.concatenate([jnp.max(el, axis=0, keepdims=True), zpad], axis=-1)
    merp = jnp.concatenate([jnp.max(er, axis=0, keepdims=True), zpad], axis=-1)
    cel_ref[...] = jnp.maximum(cel_ref[...], melp)
    cer_ref[...] = jnp.maximum(cer_ref[...], merp)


def _tc1(x, proj_W, proj_b, fc0_W, attn_l0, attn_r0, res0_W):
    rep = lambda shape: pl.BlockSpec(shape, lambda i: tuple(0 for _ in shape))
    return pl.pallas_call(
        _tc1_body,
        grid=(GRID,),
        in_specs=[
            pl.BlockSpec((NB, D), lambda i: (i, 0)),
            rep((D, D)), rep((1, D)), rep((D, H0 * D)),
            rep((1, H0 * D)), rep((1, H0 * D)), rep((D, H0 * D)),
        ],
        out_specs=[
            pl.BlockSpec((H0, NB, D), lambda i: (0, i, 0)),
            pl.BlockSpec((NB, H0), lambda i: (i, 0)),
            pl.BlockSpec((NB, H0), lambda i: (i, 0)),
            pl.BlockSpec((H0, NB, D), lambda i: (0, i, 0)),
            rep((1, D)), rep((1, 16)), rep((1, 16)),
        ],
        out_shape=[
            jax.ShapeDtypeStruct((H0, N, D), jnp.float32),
            jax.ShapeDtypeStruct((N, H0), jnp.float32),
            jax.ShapeDtypeStruct((N, H0), jnp.float32),
            jax.ShapeDtypeStruct((H0, N, D), jnp.float32),
            jax.ShapeDtypeStruct((1, D), jnp.float32),
            jax.ShapeDtypeStruct((1, 16), jnp.float32),
            jax.ShapeDtypeStruct((1, 16), jnp.float32),
        ],
    )(x, proj_W, proj_b, fc0_W, attn_l0, attn_r0, res0_W)


@functools.lru_cache(maxsize=None)
def _sc_mesh():
    return plsc.VectorSubcoreMesh(core_axis_name="c", subcore_axis_name="s")


def _exp16(x):
    y = x * 1.4426950408889634
    k = y.astype(jnp.int32)
    k = jnp.where(y < k.astype(jnp.float32), k - 1, k)
    f = y - k.astype(jnp.float32)
    p = jnp.full((16,), 1.8775767e-3, jnp.float32)
    for coef in (8.9893397e-3, 5.5826318e-2, 2.4015361e-1, 6.9315308e-1, 1.0):
        p = p * f + coef
    kc = jnp.maximum(k, -126)
    scale = lax.bitcast_convert_type((kc + 127) << 23, jnp.float32)
    return p * scale


def _zero_rows(buf, nrows):
    @pl.loop(0, nrows)
    def _(j):
        for r in range(D // 16):
            buf[j, pl.ds(r * 16, 16)] = jnp.zeros((16,), jnp.float32)


def _zero_vec(buf, nelem):
    @pl.loop(0, nelem // 16)
    def _(j):
        buf[pl.ds(j * 16, 16)] = jnp.zeros((16,), jnp.float32)


def _zero_vec_i32(buf, nelem):
    @pl.loop(0, nelem // 16)
    def _(j):
        buf[pl.ds(j * 16, 16)] = jnp.zeros((16,), jnp.int32)


def _idx_prefetch(buf, src_hbm, dst_hbm, base):
    (srcv, dstv, ridx, lidx, didx, dscat, rows, elg, erg, eev,
     sm, scs, isem) = buf
    pltpu.async_copy(src_hbm.at[pl.ds(base, CH)], srcv, isem)
    pltpu.async_copy(dst_hbm.at[pl.ds(base, CH)], dstv, isem)


def _idx_drain(buf, src_hbm, dst_hbm):
    (srcv, dstv, ridx, lidx, didx, dscat, rows, elg, erg, eev,
     sm, scs, isem) = buf
    pltpu.make_async_copy(src_hbm.at[pl.ds(0, CH)], srcv, isem).wait()
    pltpu.make_async_copy(dst_hbm.at[pl.ds(0, CH)], dstv, isem).wait()


def _chunk_start(buf, acc_sh, tbl_hbm, elf_hbm, erf_hbm, src_hbm, dst_hbm,
                 base, hoff, emul, eoff):
    (srcv, dstv, ridx, lidx, didx, dscat, rows, elg, erg, eev,
     sm, scs, isem) = buf
    _idx_drain(buf, src_hbm, dst_hbm)
    pltpu.make_async_copy(rows, acc_sh.at[dscat], scs).wait()
    for g in range(CH // 16):
        sl = pl.ds(g * 16, 16)
        sv = srcv[sl]
        dv = dstv[sl]
        ridx[sl] = sv + hoff
        lidx[sl] = sv * emul + eoff
        didx[sl] = dv * emul + eoff
        dscat[sl] = dv
    pltpu.async_copy(tbl_hbm.at[ridx], rows, sm)
    pltpu.async_copy(elf_hbm.at[lidx], elg, sm)
    pltpu.async_copy(erf_hbm.at[didx], erg, sm)
    _idx_prefetch(buf, src_hbm, dst_hbm, base + 2 * CH)


def _chunk_finish(buf, tbl_hbm, elf_hbm, erf_hbm, acc_sh, denp, cc, masks):
    (srcv, dstv, ridx, lidx, didx, dscat, rows, elg, erg, eev,
     sm, scs, isem) = buf
    pltpu.make_async_copy(tbl_hbm.at[ridx], rows, sm).wait()
    pltpu.make_async_copy(elf_hbm.at[lidx], elg, sm).wait()
    pltpu.make_async_copy(erf_hbm.at[didx], erg, sm).wait()
    for g in range(CH // 16):
        sl = pl.ds(g * 16, 16)
        e = elg[sl] + erg[sl]
        e = jnp.where(e >= 0.0, e, e * 0.2)
        ee = _exp16(jnp.maximum(e - cc, -100.0))
        eev[sl] = ee
        dv = dscat[sl]
        for k in range(16):
            plsc.addupdate_scatter(denp, [dv], ee, mask=masks[k])

    @pl.loop(0, CH, unroll=4)
    def _(j):
        sc = eev[pl.ds(j, 16)][0]
        for r in range(D // 16):
            sl = pl.ds(r * 16, 16)
            rows[j, sl] = rows[j, sl] * sc

    pltpu.async_copy(rows, acc_sh.at[dscat], scs, add=True)


def _edge_pipeline(bufs, tbl_hbm, elf_hbm, erf_hbm, src_hbm, dst_hbm, ebase,
                   hoff, emul, eoff, acc_sh, denp, cc, masks):
    nch = NCHK
    start = lambda k, b: _chunk_start(bufs[b], acc_sh, tbl_hbm, elf_hbm,
                                      erf_hbm, src_hbm, dst_hbm,
                                      ebase + k * CH, hoff, emul, eoff)
    finish = lambda b: _chunk_finish(bufs[b], tbl_hbm, elf_hbm, erf_hbm,
                                     acc_sh, denp, cc, masks)
    for b in range(2):
        (_, _, _, _, _, dscat, rows, _, _, _, _, scs, _) = bufs[b]
        _zero_vec_i32(dscat, CH)
        _zero_rows(rows, CH)
        pltpu.async_copy(rows, acc_sh.at[dscat], scs, add=True)
        _idx_prefetch(bufs[b], src_hbm, dst_hbm, ebase + b * CH)
    start(0, 0)
    if nch % 2 == 0:
        @pl.loop(0, nch - 2, step=2)
        def _(ci):
            for b in range(2):
                start(ci + b + 1, 1 - b)
                finish(b)

        start(nch - 1, 1)
        finish(0)
        finish(1)
    else:
        @pl.loop(0, nch - 1, step=2)
        def _(ci):
            for b in range(2):
                start(ci + b + 1, 1 - b)
                finish(b)

        finish(0)
    for b in range(2):
        (_, _, _, _, _, dscat, rows, _, _, _, _, scs, _) = bufs[b]
        pltpu.make_async_copy(rows, acc_sh.at[dscat], scs).wait()
        _idx_drain(bufs[b], src_hbm, dst_hbm)


def _reduce_divide_writeout(coff, s, acc_sh, den_sh, denp, dtmp, dsum, dbuf,
                            out_view):
    pltpu.sync_copy(denp, den_sh.at[pl.ds(coff + s * NPAD, NPAD)])
    plsc.subcore_barrier()
    _zero_vec(dsum, NPT)
    for k in range(NTILES):
        pltpu.sync_copy(den_sh.at[pl.ds(coff + k * NPAD + s * NPT, NPT)],
                        dtmp)

        @pl.loop(0, NPT // 16)
        def _(i):
            sl = pl.ds(i * 16, 16)
            dsum[sl] = dsum[sl] + dtmp[sl]

    @pl.loop(0, NPT // 16)
    def _(i):
        sl = pl.ds(i * 16, 16)
        dsum[sl] = 1.0 / (dsum[sl] + 1e-9)

    @pl.loop(0, NPT // ZCH)
    def _(k):
        pltpu.sync_copy(acc_sh.at[pl.ds(s * NPT + k * ZCH, ZCH)], dbuf)

        @pl.loop(0, ZCH)
        def _(j):
            dinv = dsum[pl.ds(k * ZCH + j, 16)][0]
            for r in range(D // 16):
                sl = pl.ds(r * 16, 16)
                dbuf[j, sl] = dbuf[j, sl] * dinv

        pltpu.sync_copy(dbuf, out_view.at[pl.ds(s * NPT + k * ZCH, ZCH)])


def _sc_l0_body(tbl_hbm, elf_hbm, erf_hbm, cel_hbm, cer_hbm, src_hbm, dst_hbm,
                out_hbm, den_sh, acc_sh, celv, cerv,
                srcv0, dstv0, ridx0, lidx0, didx0, dscat0, rows0, elg0,
                erg0, eev0, sm0, scs0, isem0,
                srcv1, dstv1, ridx1, lidx1, didx1, dscat1, rows1, elg1,
                erg1, eev1, sm1, scs1, isem1,
                denp, dtmp, dsum, dbuf):
    bufs = [(srcv0, dstv0, ridx0, lidx0, didx0, dscat0, rows0, elg0, erg0,
             eev0, sm0, scs0, isem0),
            (srcv1, dstv1, ridx1, lidx1, didx1, dscat1, rows1, elg1, erg1,
             eev1, sm1, scs1, isem1)]
    c = lax.axis_index("c")
    s = lax.axis_index("s")
    pltpu.sync_copy(cel_hbm, celv)
    pltpu.sync_copy(cer_hbm, cerv)
    ccv = celv[...] + cerv[...]
    lanes = lax.iota(jnp.int32, 16)
    masks = [lanes == k for k in range(16)]
    for hp in range(H0 // 2):
        h = c * (H0 // 2) + hp
        cc = jnp.where(c == 0, ccv[hp], ccv[H0 // 2 + hp])
        _zero_rows(dbuf, ZCH)

        @pl.loop(0, NPT // ZCH)
        def _(k):
            pltpu.sync_copy(dbuf, acc_sh.at[pl.ds(s * NPT + k * ZCH, ZCH)])
        _zero_vec(denp, NPAD)
        plsc.subcore_barrier()

        _edge_pipeline(bufs, tbl_hbm, elf_hbm, erf_hbm, src_hbm, dst_hbm,
                       s * EPT, h * N, H0, h, acc_sh, denp, cc, masks)

        plsc.subcore_barrier()
        _reduce_divide_writeout(c * (NTILES * NPAD), s, acc_sh, den_sh, denp,
                                dtmp, dsum, dbuf, out_hbm.at[h])
        plsc.subcore_barrier()


def _sc_l1_body(tbl_hbm, elf_hbm, erf_hbm, cel_hbm, cer_hbm, src_hbm, dst_hbm,
                out_hbm, den_sh, acc_sh, celv, cerv,
                srcv0, dstv0, ridx0, lidx0, didx0, dscat0, rows0, elg0,
                erg0, eev0, sm0, scs0, isem0,
                srcv1, dstv1, ridx1, lidx1, didx1, dscat1, rows1, elg1,
                erg1, eev1, sm1, scs1, isem1,
                denp, dtmp, dsum, dbuf):
    bufs = [(srcv0, dstv0, ridx0, lidx0, didx0, dscat0, rows0, elg0, erg0,
             eev0, sm0, scs0, isem0),
            (srcv1, dstv1, ridx1, lidx1, didx1, dscat1, rows1, elg1, erg1,
             eev1, sm1, scs1, isem1)]
    c = lax.axis_index("c")
    s = lax.axis_index("s")

    @pl.when(c == 0)
    def _():
        pltpu.sync_copy(cel_hbm, celv)
        pltpu.sync_copy(cer_hbm, cerv)
        cc = (celv[...] + cerv[...])[0]
        lanes = lax.iota(jnp.int32, 16)
        masks = [lanes == k for k in range(16)]
        _zero_rows(dbuf, ZCH)

        @pl.loop(0, NPT // ZCH)
        def _(k):
            pltpu.sync_copy(dbuf, acc_sh.at[pl.ds(s * NPT + k * ZCH, ZCH)])
        _zero_vec(denp, NPAD)
        plsc.subcore_barrier()

        _edge_pipeline(bufs, tbl_hbm, elf_hbm, erf_hbm, src_hbm, dst_hbm,
                       s * EPT, 0, 1, 0, acc_sh, denp, cc, masks)

        plsc.subcore_barrier()
        _reduce_divide_writeout(0, s, acc_sh, den_sh, denp, dtmp, dsum, dbuf,
                                out_hbm)


_SC_SCRATCH = [
    pltpu.VMEM_SHARED((NPAD, D), jnp.float32),
    pltpu.VMEM((16,), jnp.float32),
    pltpu.VMEM((16,), jnp.float32),
] + 2 * [
    pltpu.VMEM((CH,), jnp.int32),
    pltpu.VMEM((CH,), jnp.int32),
    pltpu.VMEM((CH,), jnp.int32),
    pltpu.VMEM((CH,), jnp.int32),
    pltpu.VMEM((CH,), jnp.int32),
    pltpu.VMEM((CH,), jnp.int32),
    pltpu.VMEM((CH, D), jnp.float32),
    pltpu.VMEM((CH,), jnp.float32),
    pltpu.VMEM((CH,), jnp.float32),
    pltpu.VMEM((CH + 16,), jnp.float32),
    pltpu.SemaphoreType.DMA,
    pltpu.SemaphoreType.DMA,
    pltpu.SemaphoreType.DMA,
] + [
    pltpu.VMEM((NPAD,), jnp.float32),
    pltpu.VMEM((NPT,), jnp.float32),
    pltpu.VMEM((NPT + 16,), jnp.float32),
    pltpu.VMEM((ZCH, D), jnp.float32),
]


@functools.lru_cache(maxsize=None)
def _sc_l0_kernel():
    return pl.kernel(
        _sc_l0_body,
        out_type=(jax.ShapeDtypeStruct((H0, NPAD, D), jnp.float32),
                  jax.ShapeDtypeStruct((2 * NTILES * NPAD,), jnp.float32)),
        mesh=_sc_mesh(),
        scratch_types=list(_SC_SCRATCH),
        compiler_params=pltpu.CompilerParams(needs_layout_passes=False),
    )


def _sc_l0(*args):
    return _sc_l0_kernel()(*args)[0]


@functools.lru_cache(maxsize=None)
def _sc_l1_kernel():
    return pl.kernel(
        _sc_l1_body,
        out_type=(jax.ShapeDtypeStruct((NPAD, D), jnp.float32),
                  jax.ShapeDtypeStruct((2 * NTILES * NPAD,), jnp.float32)),
        mesh=_sc_mesh(),
        scratch_types=list(_SC_SCRATCH),
        compiler_params=pltpu.CompilerParams(needs_layout_passes=False),
    )


def _sc_l1(*args):
    return _sc_l1_kernel()(*args)[0]


def _tc2_body(acc_ref, res_ref, b0_ref, fw1_ref, al1_ref, ar1_ref,
              h1_ref, tbl1_ref, el1_ref, er1_ref, cs1_ref, cel_ref, cer_ref):
    i = pl.program_id(0)
    acc = 0.0
    for k in range(H0):
        gat = (acc_ref[k] + res_ref[k]
               + b0_ref[:, k * D:(k + 1) * D])
        mu = jnp.mean(gat, axis=-1, keepdims=True)
        var = jnp.mean((gat - mu) ** 2, axis=-1, keepdims=True)
        acc = acc + (gat - mu) / jnp.sqrt(var + 1e-5)
    h1 = acc * (1.0 / H0)
    h1_ref[...] = h1
    feat1 = jnp.dot(h1, fw1_ref[...], preferred_element_type=jnp.float32)
    el1 = jnp.sum(feat1 * al1_ref[...], axis=-1, keepdims=True)
    er1 = jnp.sum(feat1 * ar1_ref[...], axis=-1, keepdims=True)
    el1_ref[...] = el1
    er1_ref[...] = er1
    tbl1_ref[...] = feat1

    @pl.when(i == 0)
    def _():
        cs1_ref[...] = jnp.zeros_like(cs1_ref)
        cel_ref[...] = jnp.full_like(cel_ref, NEG)
        cer_ref[...] = jnp.full_like(cer_ref, NEG)

    cs1_ref[...] += jnp.sum(h1, axis=0, keepdims=True)
    cel_ref[...] = jnp.maximum(cel_ref[...],
                               jnp.full((1, 16), jnp.max(el1), jnp.float32))
    cer_ref[...] = jnp.maximum(cer_ref[...],
                               jnp.full((1, 16), jnp.max(er1), jnp.float32))


def _tc2(acc0, res0, bias0, fc1_W, attn_l1, attn_r1):
    rep = lambda shape: pl.BlockSpec(shape, lambda i: tuple(0 for _ in shape))
    return pl.pallas_call(
        _tc2_body,
        grid=(GRID,),
        in_specs=[
            pl.BlockSpec((H0, NB, D), lambda i: (0, i, 0)),
            pl.BlockSpec((H0, NB, D), lambda i: (0, i, 0)),
            rep((1, H0 * D)), rep((D, D)), rep((1, D)), rep((1, D)),
        ],
        out_specs=[
            pl.BlockSpec((NB, D), lambda i: (i, 0)),
            pl.BlockSpec((NB, D), lambda i: (i, 0)),
            pl.BlockSpec((NB, 1), lambda i: (i, 0)),
            pl.BlockSpec((NB, 1), lambda i: (i, 0)),
            rep((1, D)), rep((1, 16)), rep((1, 16)),
        ],
        out_shape=[
            jax.ShapeDtypeStruct((N, D), jnp.float32),
            jax.ShapeDtypeStruct((N, D), jnp.float32),
            jax.ShapeDtypeStruct((N, 1), jnp.float32),
            jax.ShapeDtypeStruct((N, 1), jnp.float32),
            jax.ShapeDtypeStruct((1, D), jnp.float32),
            jax.ShapeDtypeStruct((1, 16), jnp.float32),
            jax.ShapeDtypeStruct((1, 16), jnp.float32),
        ],
    )(acc0, res0, bias0, fc1_W, attn_l1, attn_r1)


def _tc3_body(acc_ref, h1_ref, b1_ref, hg0_ref, cs1_ref, gl0w_ref, gl0b_ref,
              gl1w_ref, gl1b_ref, m0w_ref, m0b_ref, m1w_ref, m1b_ref,
              m2w_ref, m2b_ref, out_ref, cs2_ref):
    i = pl.program_id(0)
    gat = acc_ref[...] + h1_ref[...] + b1_ref[...]
    mu = jnp.mean(gat, axis=-1, keepdims=True)
    var = jnp.mean((gat - mu) ** 2, axis=-1, keepdims=True)
    h2 = (gat - mu) / jnp.sqrt(var + 1e-5)

    @pl.when(i == 0)
    def _():
        cs2_ref[...] = jnp.zeros_like(cs2_ref)

    cs2_ref[...] += jnp.sum(h2, axis=0, keepdims=True)

    @pl.when(i == GRID - 1)
    def _():
        dot = lambda a, b: jnp.dot(a, b, preferred_element_type=jnp.float32)
        hg = (hg0_ref[...]
              + _lrelu(dot(cs1_ref[...], gl0w_ref[...]) + gl0b_ref[...], 0.01)
              + _lrelu(dot(cs2_ref[...], gl1w_ref[...]) + gl1b_ref[...], 0.01))
        hg = dot(hg, m0w_ref[...]) + m0b_ref[...]
        hg = dot(jnp.maximum(hg, 0.0), m1w_ref[...]) + m1b_ref[...]
        hg = dot(jnp.maximum(hg, 0.0), m2w_ref[...]) + m2b_ref[...]
        out_ref[...] = hg


def _tc3(acc1, h1, bias1, hg0, cs1, gl0_W, gl0_b, gl1_W, gl1_b,
         m0_W, m0_b, m1_W, m1_b, m2_W, m2_b):
    rep = lambda shape: pl.BlockSpec(shape, lambda i: tuple(0 for _ in shape))
    mlp = m0_W.shape[1]
    return pl.pallas_call(
        _tc3_body,
        grid=(GRID,),
        in_specs=[
            pl.BlockSpec((NB, D), lambda i: (i, 0)),
            pl.BlockSpec((NB, D), lambda i: (i, 0)),
            rep((1, D)), rep((1, D)), rep((1, D)),
            rep((D, D)), rep((1, D)), rep((D, D)), rep((1, D)),
            rep((D, mlp)), rep((1, mlp)), rep((mlp, mlp)), rep((1, mlp)),
            rep((mlp, mlp)), rep((1, mlp)),
        ],
        out_specs=[rep((1, mlp))],
        out_shape=[jax.ShapeDtypeStruct((1, mlp), jnp.float32)],
        scratch_shapes=[pltpu.VMEM((1, D), jnp.float32)],
    )(acc1, h1, bias1, hg0, cs1, gl0_W, gl0_b, gl1_W, gl1_b,
      m0_W, m0_b, m1_W, m1_b, m2_W, m2_b)[0]


@jax.jit
def kernel(node_features, edge_index, proj_W, proj_b, fc0_W, attn_l0, attn_r0,
           res0_W, bias0, gl0_W, gl0_b, fc1_W, attn_l1, attn_r1, bias1,
           gl1_W, gl1_b, m0_W, m0_b, m1_W, m1_b, m2_W, m2_b):
    pad = EPAD - E
    src = jnp.concatenate([edge_index[0], jnp.zeros(pad + EXTRA, jnp.int32)])
    dst = jnp.concatenate([edge_index[1], jnp.full(pad, N, jnp.int32),
                           jnp.full(EXTRA, N, jnp.int32)])
    erpad = jnp.full(16, NEG, jnp.float32)
    row = lambda v: v.reshape(1, -1)

    tbl0, elf0, erf0, res0, hg0, cel0, cer0 = _tc1(
        node_features, proj_W, row(proj_b), fc0_W,
        row(attn_l0.reshape(-1)), row(attn_r0.reshape(-1)), res0_W)

    acc0 = _sc_l0(tbl0.reshape(H0 * N, D), elf0.reshape(-1),
                  jnp.concatenate([erf0.reshape(-1), erpad]),
                  cel0.reshape(-1), cer0.reshape(-1), src, dst)

    h1, tbl1, el1, er1, cs1, cel1, cer1 = _tc2(
        acc0.reshape(H0, NPAD, D), res0, row(bias0), fc1_W,
        row(attn_l1.reshape(-1)), row(attn_r1.reshape(-1)))

    acc1 = _sc_l1(tbl1, el1.reshape(-1),
                  jnp.concatenate([er1.reshape(-1), erpad]),
                  cel1.reshape(-1), cer1.reshape(-1), src, dst)

    return _tc3(acc1, h1, row(bias1), hg0, cs1, gl0_W, row(gl0_b),
                gl1_W, row(gl1_b), m0_W, row(m0_b), m1_W, row(m1_b),
                m2_W, row(m2_b))


# speedup vs baseline: 1.5547x; 1.0024x over previous
import functools

import jax
import jax.numpy as jnp
from jax import lax
from jax.experimental import pallas as pl
from jax.experimental.pallas import tpu as pltpu
from jax.experimental.pallas import tpu_sc as plsc

N = 10000
E = 320000
D = 128
H0 = 8
NB = 400
GRID = N // NB
NTILES = 16
NPAD = 10240
NPT = NPAD // NTILES
ZCH = 128
CH = 64
EPAD = 320512
EPT = EPAD // NTILES
NCHK = EPT // CH
EXTRA = 2 * CH
NEG = -1e30


def _lrelu(x, slope):
    return jnp.where(x >= 0.0, x, x * slope)


def _tc1_body(x_ref, pw_ref, pb_ref, fw_ref, al_ref, ar_ref, rw_ref,
              tbl_ref, elf_ref, erf_ref, res_ref, hg_ref, cel_ref, cer_ref):
    i = pl.program_id(0)
    x = x_ref[...]
    h = jnp.dot(x, pw_ref[...], preferred_element_type=jnp.float32) + pb_ref[...]
    els = []
    ers = []
    for k in range(H0):
        fk = jnp.dot(h, fw_ref[:, k * D:(k + 1) * D],
                     preferred_element_type=jnp.float32)
        tbl_ref[k] = fk
        res_ref[k] = jnp.dot(h, rw_ref[:, k * D:(k + 1) * D],
                             preferred_element_type=jnp.float32)
        els.append(jnp.sum(fk * al_ref[:, k * D:(k + 1) * D], axis=-1,
                           keepdims=True))
        ers.append(jnp.sum(fk * ar_ref[:, k * D:(k + 1) * D], axis=-1,
                           keepdims=True))
    el = jnp.concatenate(els, axis=-1)
    er = jnp.concatenate(ers, axis=-1)
    elf_ref[...] = el
    erf_ref[...] = er

    @pl.when(i == 0)
    def _():
        hg_ref[...] = jnp.zeros_like(hg_ref)
        cel_ref[...] = jnp.full_like(cel_ref, NEG)
        cer_ref[...] = jnp.full_like(cer_ref, NEG)

    hg_ref[...] += jnp.sum(h, axis=0, keepdims=True)
    zpad = jnp.full((1, 8), NEG, jnp.float32)
    melp = jnp.concatenate([jnp.max(el, axis=0, keepdims=True), zpad], axis=-1)
    merp = jnp.concatenate([jnp.max(er, axis=0, keepdims=True), zpad], axis=-1)
    cel_ref[...] = jnp.maximum(cel_ref[...], melp)
    cer_ref[...] = jnp.maximum(cer_ref[...], merp)


def _tc1(x, proj_W, proj_b, fc0_W, attn_l0, attn_r0, res0_W):
    rep = lambda shape: pl.BlockSpec(shape, lambda i: tuple(0 for _ in shape))
    return pl.pallas_call(
        _tc1_body,
        grid=(GRID,),
        in_specs=[
            pl.BlockSpec((NB, D), lambda i: (i, 0)),
            rep((D, D)), rep((1, D)), rep((D, H0 * D)),
            rep((1, H0 * D)), rep((1, H0 * D)), rep((D, H0 * D)),
        ],
        out_specs=[
            pl.BlockSpec((H0, NB, D), lambda i: (0, i, 0)),
            pl.BlockSpec((NB, H0), lambda i: (i, 0)),
            pl.BlockSpec((NB, H0), lambda i: (i, 0)),
            pl.BlockSpec((H0, NB, D), lambda i: (0, i, 0)),
            rep((1, D)), rep((1, 16)), rep((1, 16)),
        ],
        out_shape=[
            jax.ShapeDtypeStruct((H0, N, D), jnp.float32),
            jax.ShapeDtypeStruct((N, H0), jnp.float32),
            jax.ShapeDtypeStruct((N, H0), jnp.float32),
            jax.ShapeDtypeStruct((H0, N, D), jnp.float32),
            jax.ShapeDtypeStruct((1, D), jnp.float32),
            jax.ShapeDtypeStruct((1, 16), jnp.float32),
            jax.ShapeDtypeStruct((1, 16), jnp.float32),
        ],
    )(x, proj_W, proj_b, fc0_W, attn_l0, attn_r0, res0_W)


@functools.lru_cache(maxsize=None)
def _sc_mesh():
    return plsc.VectorSubcoreMesh(core_axis_name="c", subcore_axis_name="s")


def _exp16(x):
    y = x * 1.4426950408889634
    k = y.astype(jnp.int32)
    k = jnp.where(y < k.astype(jnp.float32), k - 1, k)
    f = y - k.astype(jnp.float32)
    p = jnp.full((16,), 1.8775767e-3, jnp.float32)
    for coef in (8.9893397e-3, 5.5826318e-2, 2.4015361e-1, 6.9315308e-1, 1.0):
        p = p * f + coef
    kc = jnp.maximum(k, -126)
    scale = lax.bitcast_convert_type((kc + 127) << 23, jnp.float32)
    return p * scale


def _zero_rows(buf, nrows):
    @pl.loop(0, nrows)
    def _(j):
        for r in range(D // 16):
            buf[j, pl.ds(r * 16, 16)] = jnp.zeros((16,), jnp.float32)


def _zero_vec(buf, nelem):
    @pl.loop(0, nelem // 16)
    def _(j):
        buf[pl.ds(j * 16, 16)] = jnp.zeros((16,), jnp.float32)


def _zero_vec_i32(buf, nelem):
    @pl.loop(0, nelem // 16)
    def _(j):
        buf[pl.ds(j * 16, 16)] = jnp.zeros((16,), jnp.int32)


def _idx_prefetch(buf, src_hbm, dst_hbm, base):
    (srcv, dstv, ridx, lidx, didx, dscat, rows, elg, erg, eev,
     sm, els, scs, isem) = buf
    pltpu.async_copy(src_hbm.at[pl.ds(base, CH)], srcv, isem)
    pltpu.async_copy(dst_hbm.at[pl.ds(base, CH)], dstv, isem)


def _idx_drain(buf, src_hbm, dst_hbm):
    (srcv, dstv, ridx, lidx, didx, dscat, rows, elg, erg, eev,
     sm, els, scs, isem) = buf
    pltpu.make_async_copy(src_hbm.at[pl.ds(0, CH)], srcv, isem).wait()
    pltpu.make_async_copy(dst_hbm.at[pl.ds(0, CH)], dstv, isem).wait()


def _chunk_start(buf, acc_sh, tbl_hbm, elf_hbm, erf_hbm, src_hbm, dst_hbm,
                 base, hoff, emul, eoff):
    (srcv, dstv, ridx, lidx, didx, dscat, rows, elg, erg, eev,
     sm, els, scs, isem) = buf
    _idx_drain(buf, src_hbm, dst_hbm)
    pltpu.make_async_copy(rows, acc_sh.at[dscat], scs).wait()
    for g in range(CH // 16):
        sl = pl.ds(g * 16, 16)
        sv = srcv[sl]
        dv = dstv[sl]
        ridx[sl] = sv + hoff
        lidx[sl] = sv * emul + eoff
        didx[sl] = dv * emul + eoff
        dscat[sl] = dv
    pltpu.async_copy(tbl_hbm.at[ridx], rows, sm)
    pltpu.async_copy(elf_hbm.at[lidx], elg, els)
    pltpu.async_copy(erf_hbm.at[didx], erg, els)
    _idx_prefetch(buf, src_hbm, dst_hbm, base + 2 * CH)


def _chunk_finish(buf, tbl_hbm, elf_hbm, erf_hbm, acc_sh, denp, cc, masks):
    (srcv, dstv, ridx, lidx, didx, dscat, rows, elg, erg, eev,
     sm, els, scs, isem) = buf
    pltpu.make_async_copy(elf_hbm.at[lidx], elg, els).wait()
    pltpu.make_async_copy(erf_hbm.at[didx], erg, els).wait()
    for g in range(CH // 16):
        sl = pl.ds(g * 16, 16)
        e = elg[sl] + erg[sl]
        e = jnp.where(e >= 0.0, e, e * 0.2)
        ee = _exp16(jnp.maximum(e - cc, -100.0))
        eev[sl] = ee
        dv = dscat[sl]
        for k in range(16):
            plsc.addupdate_scatter(denp, [dv], ee, mask=masks[k])
    pltpu.make_async_copy(tbl_hbm.at[ridx], rows, sm).wait()

    @pl.loop(0, CH, unroll=8)
    def _(j):
        sc = eev[pl.ds(j, 16)][0]
        for r in range(D // 16):
            sl = pl.ds(r * 16, 16)
            rows[j, sl] = rows[j, sl] * sc

    pltpu.async_copy(rows, acc_sh.at[dscat], scs, add=True)


def _edge_pipeline(bufs, tbl_hbm, elf_hbm, erf_hbm, src_hbm, dst_hbm, ebase,
                   hoff, emul, eoff, acc_sh, denp, cc, masks):
    nch = NCHK
    start = lambda k, b: _chunk_start(bufs[b], acc_sh, tbl_hbm, elf_hbm,
                                      erf_hbm, src_hbm, dst_hbm,
                                      ebase + k * CH, hoff, emul, eoff)
    finish = lambda b: _chunk_finish(bufs[b], tbl_hbm, elf_hbm, erf_hbm,
                                     acc_sh, denp, cc, masks)
    for b in range(2):
        (_, _, _, _, _, dscat, rows, _, _, _, _, _, scs, _) = bufs[b]
        _zero_vec_i32(dscat, CH)
        _zero_rows(rows, CH)
        pltpu.async_copy(rows, acc_sh.at[dscat], scs, add=True)
        _idx_prefetch(bufs[b], src_hbm, dst_hbm, ebase + b * CH)
    start(0, 0)
    if nch % 2 == 0:
        @pl.loop(0, nch - 2, step=2)
        def _(ci):
            for b in range(2):
                start(ci + b + 1, 1 - b)
                finish(b)

        start(nch - 1, 1)
        finish(0)
        finish(1)
    else:
        @pl.loop(0, nch - 1, step=2)
        def _(ci):
            for b in range(2):
                start(ci + b + 1, 1 - b)
                finish(b)

        finish(0)
    for b in range(2):
        (_, _, _, _, _, dscat, rows, _, _, _, _, _, scs, _) = bufs[b]
        pltpu.make_async_copy(rows, acc_sh.at[dscat], scs).wait()
        _idx_drain(bufs[b], src_hbm, dst_hbm)


def _reduce_divide_writeout(coff, s, acc_sh, den_sh, denp, dtmp, dsum, dbuf,
                            out_view):
    pltpu.sync_copy(denp, den_sh.at[pl.ds(coff + s * NPAD, NPAD)])
    plsc.subcore_barrier()
    _zero_vec(dsum, NPT)
    for k in range(NTILES):
        pltpu.sync_copy(den_sh.at[pl.ds(coff + k * NPAD + s * NPT, NPT)],
                        dtmp)

        @pl.loop(0, NPT // 16)
        def _(i):
            sl = pl.ds(i * 16, 16)
            dsum[sl] = dsum[sl] + dtmp[sl]

    @pl.loop(0, NPT // 16)
    def _(i):
        sl = pl.ds(i * 16, 16)
        dsum[sl] = 1.0 / (dsum[sl] + 1e-9)

    @pl.loop(0, NPT // ZCH)
    def _(k):
        pltpu.sync_copy(acc_sh.at[pl.ds(s * NPT + k * ZCH, ZCH)], dbuf)

        @pl.loop(0, ZCH)
        def _(j):
            dinv = dsum[pl.ds(k * ZCH + j, 16)][0]
            for r in range(D // 16):
                sl = pl.ds(r * 16, 16)
                dbuf[j, sl] = dbuf[j, sl] * dinv

        pltpu.sync_copy(dbuf, out_view.at[pl.ds(s * NPT + k * ZCH, ZCH)])


def _sc_l0_body(tbl_hbm, elf_hbm, erf_hbm, cel_hbm, cer_hbm, src_hbm, dst_hbm,
                out_hbm, den_sh, acc_sh, celv, cerv,
                srcv0, dstv0, ridx0, lidx0, didx0, dscat0, rows0, elg0,
                erg0, eev0, sm0, els0, scs0, isem0,
                srcv1, dstv1, ridx1, lidx1, didx1, dscat1, rows1, elg1,
                erg1, eev1, sm1, els1, scs1, isem1,
                denp, dtmp, dsum, dbuf):
    bufs = [(srcv0, dstv0, ridx0, lidx0, didx0, dscat0, rows0, elg0, erg0,
             eev0, sm0, els0, scs0, isem0),
            (srcv1, dstv1, ridx1, lidx1, didx1, dscat1, rows1, elg1, erg1,
             eev1, sm1, els1, scs1, isem1)]
    c = lax.axis_index("c")
    s = lax.axis_index("s")
    pltpu.sync_copy(cel_hbm, celv)
    pltpu.sync_copy(cer_hbm, cerv)
    ccv = celv[...] + cerv[...]
    lanes = lax.iota(jnp.int32, 16)
    masks = [lanes == k for k in range(16)]
    for hp in range(H0 // 2):
        h = c * (H0 // 2) + hp
        cc = jnp.where(c == 0, ccv[hp], ccv[H0 // 2 + hp])
        _zero_rows(dbuf, ZCH)

        @pl.loop(0, NPT // ZCH)
        def _(k):
            pltpu.sync_copy(dbuf, acc_sh.at[pl.ds(s * NPT + k * ZCH, ZCH)])
        _zero_vec(denp, NPAD)
        plsc.subcore_barrier()

        _edge_pipeline(bufs, tbl_hbm, elf_hbm, erf_hbm, src_hbm, dst_hbm,
                       s * EPT, h * N, H0, h, acc_sh, denp, cc, masks)

        plsc.subcore_barrier()
        _reduce_divide_writeout(c * (NTILES * NPAD), s, acc_sh, den_sh, denp,
                                dtmp, dsum, dbuf, out_hbm.at[h])
        plsc.subcore_barrier()


def _sc_l1_body(tbl_hbm, elf_hbm, erf_hbm, cel_hbm, cer_hbm, src_hbm, dst_hbm,
                out_hbm, den_sh, acc_sh, celv, cerv,
                srcv0, dstv0, ridx0, lidx0, didx0, dscat0, rows0, elg0,
                erg0, eev0, sm0, els0, scs0, isem0,
                srcv1, dstv1, ridx1, lidx1, didx1, dscat1, rows1, elg1,
                erg1, eev1, sm1, els1, scs1, isem1,
                denp, dtmp, dsum, dbuf):
    bufs = [(srcv0, dstv0, ridx0, lidx0, didx0, dscat0, rows0, elg0, erg0,
             eev0, sm0, els0, scs0, isem0),
            (srcv1, dstv1, ridx1, lidx1, didx1, dscat1, rows1, elg1, erg1,
             eev1, sm1, els1, scs1, isem1)]
    c = lax.axis_index("c")
    s = lax.axis_index("s")

    @pl.when(c == 0)
    def _():
        pltpu.sync_copy(cel_hbm, celv)
        pltpu.sync_copy(cer_hbm, cerv)
        cc = (celv[...] + cerv[...])[0]
        lanes = lax.iota(jnp.int32, 16)
        masks = [lanes == k for k in range(16)]
        _zero_rows(dbuf, ZCH)

        @pl.loop(0, NPT // ZCH)
        def _(k):
            pltpu.sync_copy(dbuf, acc_sh.at[pl.ds(s * NPT + k * ZCH, ZCH)])
        _zero_vec(denp, NPAD)
        plsc.subcore_barrier()

        _edge_pipeline(bufs, tbl_hbm, elf_hbm, erf_hbm, src_hbm, dst_hbm,
                       s * EPT, 0, 1, 0, acc_sh, denp, cc, masks)

        plsc.subcore_barrier()
        _reduce_divide_writeout(0, s, acc_sh, den_sh, denp, dtmp, dsum, dbuf,
                                out_hbm)


_SC_SCRATCH = [
    pltpu.VMEM_SHARED((NPAD, D), jnp.float32),
    pltpu.VMEM((16,), jnp.float32),
    pltpu.VMEM((16,), jnp.float32),
] + 2 * [
    pltpu.VMEM((CH,), jnp.int32),
    pltpu.VMEM((CH,), jnp.int32),
    pltpu.VMEM((CH,), jnp.int32),
    pltpu.VMEM((CH,), jnp.int32),
    pltpu.VMEM((CH,), jnp.int32),
    pltpu.VMEM((CH,), jnp.int32),
    pltpu.VMEM((CH, D), jnp.float32),
    pltpu.VMEM((CH,), jnp.float32),
    pltpu.VMEM((CH,), jnp.float32),
    pltpu.VMEM((CH + 16,), jnp.float32),
    pltpu.SemaphoreType.DMA,
    pltpu.SemaphoreType.DMA,
    pltpu.SemaphoreType.DMA,
    pltpu.SemaphoreType.DMA,
] + [
    pltpu.VMEM((NPAD,), jnp.float32),
    pltpu.VMEM((NPT,), jnp.float32),
    pltpu.VMEM((NPT + 16,), jnp.float32),
    pltpu.VMEM((ZCH, D), jnp.float32),
]


@functools.lru_cache(maxsize=None)
def _sc_l0_kernel():
    return pl.kernel(
        _sc_l0_body,
        out_type=(jax.ShapeDtypeStruct((H0, NPAD, D), jnp.float32),
                  jax.ShapeDtypeStruct((2 * NTILES * NPAD,), jnp.float32)),
        mesh=_sc_mesh(),
        scratch_types=list(_SC_SCRATCH),
        compiler_params=pltpu.CompilerParams(needs_layout_passes=False),
    )


def _sc_l0(*args):
    return _sc_l0_kernel()(*args)[0]


@functools.lru_cache(maxsize=None)
def _sc_l1_kernel():
    return pl.kernel(
        _sc_l1_body,
        out_type=(jax.ShapeDtypeStruct((NPAD, D), jnp.float32),
                  jax.ShapeDtypeStruct((2 * NTILES * NPAD,), jnp.float32)),
        mesh=_sc_mesh(),
        scratch_types=list(_SC_SCRATCH),
        compiler_params=pltpu.CompilerParams(needs_layout_passes=False),
    )


def _sc_l1(*args):
    return _sc_l1_kernel()(*args)[0]


def _tc2_body(acc_ref, res_ref, b0_ref, fw1_ref, al1_ref, ar1_ref,
              h1_ref, tbl1_ref, el1_ref, er1_ref, cs1_ref, cel_ref, cer_ref):
    i = pl.program_id(0)
    acc = 0.0
    for k in range(H0):
        gat = (acc_ref[k] + res_ref[k]
               + b0_ref[:, k * D:(k + 1) * D])
        mu = jnp.mean(gat, axis=-1, keepdims=True)
        var = jnp.mean((gat - mu) ** 2, axis=-1, keepdims=True)
        acc = acc + (gat - mu) / jnp.sqrt(var + 1e-5)
    h1 = acc * (1.0 / H0)
    h1_ref[...] = h1
    feat1 = jnp.dot(h1, fw1_ref[...], preferred_element_type=jnp.float32)
    el1 = jnp.sum(feat1 * al1_ref[...], axis=-1, keepdims=True)
    er1 = jnp.sum(feat1 * ar1_ref[...], axis=-1, keepdims=True)
    el1_ref[...] = el1
    er1_ref[...] = er1
    tbl1_ref[...] = feat1

    @pl.when(i == 0)
    def _():
        cs1_ref[...] = jnp.zeros_like(cs1_ref)
        cel_ref[...] = jnp.full_like(cel_ref, NEG)
        cer_ref[...] = jnp.full_like(cer_ref, NEG)

    cs1_ref[...] += jnp.sum(h1, axis=0, keepdims=True)
    cel_ref[...] = jnp.maximum(cel_ref[...],
                               jnp.full((1, 16), jnp.max(el1), jnp.float32))
    cer_ref[...] = jnp.maximum(cer_ref[...],
                               jnp.full((1, 16), jnp.max(er1), jnp.float32))


def _tc2(acc0, res0, bias0, fc1_W, attn_l1, attn_r1):
    rep = lambda shape: pl.BlockSpec(shape, lambda i: tuple(0 for _ in shape))
    return pl.pallas_call(
        _tc2_body,
        grid=(GRID,),
        in_specs=[
            pl.BlockSpec((H0, NB, D), lambda i: (0, i, 0)),
            pl.BlockSpec((H0, NB, D), lambda i: (0, i, 0)),
            rep((1, H0 * D)), rep((D, D)), rep((1, D)), rep((1, D)),
        ],
        out_specs=[
            pl.BlockSpec((NB, D), lambda i: (i, 0)),
            pl.BlockSpec((NB, D), lambda i: (i, 0)),
            pl.BlockSpec((NB, 1), lambda i: (i, 0)),
            pl.BlockSpec((NB, 1), lambda i: (i, 0)),
            rep((1, D)), rep((1, 16)), rep((1, 16)),
        ],
        out_shape=[
            jax.ShapeDtypeStruct((N, D), jnp.float32),
            jax.ShapeDtypeStruct((N, D), jnp.float32),
            jax.ShapeDtypeStruct((N, 1), jnp.float32),
            jax.ShapeDtypeStruct((N, 1), jnp.float32),
            jax.ShapeDtypeStruct((1, D), jnp.float32),
            jax.ShapeDtypeStruct((1, 16), jnp.float32),
            jax.ShapeDtypeStruct((1, 16), jnp.float32),
        ],
    )(acc0, res0, bias0, fc1_W, attn_l1, attn_r1)


def _tc3_body(acc_ref, h1_ref, b1_ref, hg0_ref, cs1_ref, gl0w_ref, gl0b_ref,
              gl1w_ref, gl1b_ref, m0w_ref, m0b_ref, m1w_ref, m1b_ref,
              m2w_ref, m2b_ref, out_ref, cs2_ref):
    i = pl.program_id(0)
    gat = acc_ref[...] + h1_ref[...] + b1_ref[...]
    mu = jnp.mean(gat, axis=-1, keepdims=True)
    var = jnp.mean((gat - mu) ** 2, axis=-1, keepdims=True)
    h2 = (gat - mu) / jnp.sqrt(var + 1e-5)

    @pl.when(i == 0)
    def _():
        cs2_ref[...] = jnp.zeros_like(cs2_ref)

    cs2_ref[...] += jnp.sum(h2, axis=0, keepdims=True)

    @pl.when(i == GRID - 1)
    def _():
        dot = lambda a, b: jnp.dot(a, b, preferred_element_type=jnp.float32)
        hg = (hg0_ref[...]
              + _lrelu(dot(cs1_ref[...], gl0w_ref[...]) + gl0b_ref[...], 0.01)
              + _lrelu(dot(cs2_ref[...], gl1w_ref[...]) + gl1b_ref[...], 0.01))
        hg = dot(hg, m0w_ref[...]) + m0b_ref[...]
        hg = dot(jnp.maximum(hg, 0.0), m1w_ref[...]) + m1b_ref[...]
        hg = dot(jnp.maximum(hg, 0.0), m2w_ref[...]) + m2b_ref[...]
        out_ref[...] = hg


def _tc3(acc1, h1, bias1, hg0, cs1, gl0_W, gl0_b, gl1_W, gl1_b,
         m0_W, m0_b, m1_W, m1_b, m2_W, m2_b):
    rep = lambda shape: pl.BlockSpec(shape, lambda i: tuple(0 for _ in shape))
    mlp = m0_W.shape[1]
    return pl.pallas_call(
        _tc3_body,
        grid=(GRID,),
        in_specs=[
            pl.BlockSpec((NB, D), lambda i: (i, 0)),
            pl.BlockSpec((NB, D), lambda i: (i, 0)),
            rep((1, D)), rep((1, D)), rep((1, D)),
            rep((D, D)), rep((1, D)), rep((D, D)), rep((1, D)),
            rep((D, mlp)), rep((1, mlp)), rep((mlp, mlp)), rep((1, mlp)),
            rep((mlp, mlp)), rep((1, mlp)),
        ],
        out_specs=[rep((1, mlp))],
        out_shape=[jax.ShapeDtypeStruct((1, mlp), jnp.float32)],
        scratch_shapes=[pltpu.VMEM((1, D), jnp.float32)],
    )(acc1, h1, bias1, hg0, cs1, gl0_W, gl0_b, gl1_W, gl1_b,
      m0_W, m0_b, m1_W, m1_b, m2_W, m2_b)[0]


@jax.jit
def kernel(node_features, edge_index, proj_W, proj_b, fc0_W, attn_l0, attn_r0,
           res0_W, bias0, gl0_W, gl0_b, fc1_W, attn_l1, attn_r1, bias1,
           gl1_W, gl1_b, m0_W, m0_b, m1_W, m1_b, m2_W, m2_b):
    pad = EPAD - E
    src = jnp.concatenate([edge_index[0], jnp.zeros(pad + EXTRA, jnp.int32)])
    dst = jnp.concatenate([edge_index[1], jnp.full(pad, N, jnp.int32),
                           jnp.full(EXTRA, N, jnp.int32)])
    erpad = jnp.full(16, NEG, jnp.float32)
    row = lambda v: v.reshape(1, -1)

    tbl0, elf0, erf0, res0, hg0, cel0, cer0 = _tc1(
        node_features, proj_W, row(proj_b), fc0_W,
        row(attn_l0.reshape(-1)), row(attn_r0.reshape(-1)), res0_W)

    acc0 = _sc_l0(tbl0.reshape(H0 * N, D), elf0.reshape(-1),
                  jnp.concatenate([erf0.reshape(-1), erpad]),
                  cel0.reshape(-1), cer0.reshape(-1), src, dst)

    h1, tbl1, el1, er1, cs1, cel1, cer1 = _tc2(
        acc0.reshape(H0, NPAD, D), res0, row(bias0), fc1_W,
        row(attn_l1.reshape(-1)), row(attn_r1.reshape(-1)))

    acc1 = _sc_l1(tbl1, el1.reshape(-1),
                  jnp.concatenate([er1.reshape(-1), erpad]),
                  cel1.reshape(-1), cer1.reshape(-1), src, dst)

    return _tc3(acc1, h1, row(bias1), hg0, cs1, gl0_W, row(gl0_b),
                gl1_W, row(gl1_b), m0_W, row(m0_b), m1_W, row(m1_b),
                m2_W, row(m2_b))
